# Initial kernel scaffold; baseline (speedup 1.0000x reference)
import functools

import jax
import jax.numpy as jnp
from jax import lax
from jax.experimental import pallas as pl
from jax.experimental.pallas import tpu as pltpu
from jax.experimental.pallas import tpu_sc as plsc

N = 10000
F = 128
H = 128
E = 320000

NC = 2
NS = 16
NW = NC * NS
CHUNK = 128
E_PAD = 327680
NCHUNK = E_PAD // CHUNK
CPW = NCHUNK // NW
AGG_ROWS = 10240
RPT = AGG_ROWS // NS

_mesh = plsc.VectorSubcoreMesh(
    core_axis_name="c", subcore_axis_name="s", num_cores=NC, num_subcores=NS)


@functools.partial(
    pl.kernel,
    out_type=jax.ShapeDtypeStruct((NC, AGG_ROWS, 16), jnp.float32),
    mesh=_mesh,
    scratch_types=[
        pltpu.VMEM_SHARED((AGG_ROWS, 16), jnp.float32),
        pltpu.VMEM((CPW, CHUNK), jnp.int32),
        pltpu.VMEM((CHUNK, 16), jnp.float32),
        pltpu.SemaphoreType.DMA,
    ],
)
def _deg_kernel(dstm, ones16, zeros16, out, degm, idx_all, ones_v, sem):
    c = lax.axis_index("c")
    s = lax.axis_index("s")
    wid = c * NS + s
    base = wid * CPW
    pltpu.sync_copy(zeros16.at[pl.ds(s * RPT, RPT)], degm.at[pl.ds(s * RPT, RPT)])
    pltpu.sync_copy(ones16, ones_v)
    pltpu.sync_copy(dstm.at[pl.ds(base, CPW)], idx_all)
    plsc.subcore_barrier()

    @pl.loop(0, CPW, step=16)
    def _fire_drain(g):
        @pl.loop(0, 16)
        def _fire(b):
            pltpu.async_copy(ones_v, degm.at[idx_all.at[g + b]], sem, add=True)

        @pl.loop(0, 16)
        def _drain(b):
            pltpu.make_async_copy(ones_v, degm.at[idx_all.at[0]], sem).wait()

    plsc.subcore_barrier()
    pltpu.sync_copy(degm.at[pl.ds(s * RPT, RPT)], out.at[c, pl.ds(s * RPT, RPT)])


@functools.partial(
    pl.kernel,
    out_type=jax.ShapeDtypeStruct((NC, AGG_ROWS, H), jnp.float32),
    mesh=_mesh,
    scratch_types=[
        pltpu.VMEM_SHARED((AGG_ROWS, H), jnp.float32),
        pltpu.VMEM((CPW, CHUNK), jnp.int32),
        pltpu.VMEM((CPW, CHUNK), jnp.int32),
        pltpu.VMEM((2, CHUNK, H), jnp.float32),
        pltpu.SemaphoreType.DMA,
        pltpu.SemaphoreType.DMA,
    ],
)
def _agg_kernel(xws, srcm, dstm, zer, out, agg, idxs_all, idxd_all, rows,
                sem0, sem1):
    c = lax.axis_index("c")
    s = lax.axis_index("s")
    wid = c * NS + s
    base = wid * CPW
    pltpu.sync_copy(zer.at[pl.ds(s * RPT, RPT)], agg.at[pl.ds(s * RPT, RPT)])
    pltpu.sync_copy(srcm.at[pl.ds(base, CPW)], idxs_all)
    pltpu.sync_copy(dstm.at[pl.ds(base, CPW)], idxd_all)
    plsc.subcore_barrier()

    sems = (sem0, sem1)
    pltpu.async_copy(xws.at[idxs_all.at[0]], rows.at[0], sems[0])
    pltpu.async_copy(xws.at[idxs_all.at[1]], rows.at[1], sems[1])

    @pl.loop(0, CPW, step=2)
    def _step(j):
        for b in (0, 1):
            jj = j + b
            pltpu.make_async_copy(
                xws.at[idxs_all.at[0]], rows.at[b], sems[b]).wait()
            pltpu.sync_copy(rows.at[b], agg.at[idxd_all.at[jj]], add=True)

            @pl.when(jj + 2 < CPW)
            def _prefetch():
                pltpu.async_copy(
                    xws.at[idxs_all.at[jj + 2]], rows.at[b], sems[b])

    plsc.subcore_barrier()
    pltpu.sync_copy(agg.at[pl.ds(s * RPT, RPT)], out.at[c, pl.ds(s * RPT, RPT)])


def _pre_body(x_ref, h_ref, w1_ref, w2_ref, d0_ref, d1_ref, xws_ref, dinv_ref):
    xw = (jnp.dot(x_ref[...], w1_ref[...], preferred_element_type=jnp.float32)
          + jnp.dot(h_ref[...], w2_ref[...], preferred_element_type=jnp.float32))
    dinv = lax.rsqrt(d0_ref[...] + d1_ref[...] + 1.0)
    dinv_ref[...] = dinv
    xws_ref[...] = xw * dinv


def _pre_call(x, h, w1, w2, d0, d1):
    blk = 1000
    grid = N // blk
    return pl.pallas_call(
        _pre_body,
        grid=(grid,),
        in_specs=[
            pl.BlockSpec((blk, F), lambda i: (i, 0)),
            pl.BlockSpec((blk, H), lambda i: (i, 0)),
            pl.BlockSpec((F, H), lambda i: (0, 0)),
            pl.BlockSpec((H, H), lambda i: (0, 0)),
            pl.BlockSpec((blk, 1), lambda i: (i, 0)),
            pl.BlockSpec((blk, 1), lambda i: (i, 0)),
        ],
        out_specs=[
            pl.BlockSpec((blk, H), lambda i: (i, 0)),
            pl.BlockSpec((blk, 1), lambda i: (i, 0)),
        ],
        out_shape=[
            jax.ShapeDtypeStruct((N, H), jnp.float32),
            jax.ShapeDtypeStruct((N, 1), jnp.float32),
        ],
    )(x, h, w1, w2, d0, d1)


def _post_body(a0_ref, a1_ref, xws_ref, dinv_ref, bm_ref, cm_ref, gb_ref,
               v_ref, o_ref, nh_ref):
    conv = (a0_ref[...] + a1_ref[...] + xws_ref[...]) * dinv_ref[...] + gb_ref[...]
    nh = jax.nn.sigmoid(bm_ref[...] + conv)
    nh_ref[...] = nh
    o_ref[...] = cm_ref[...] + jnp.dot(nh, v_ref[...],
                                       preferred_element_type=jnp.float32)


def _post_call(a0, a1, xws, dinv, bm, cm, gb, v):
    blk = 1000
    grid = N // blk
    return pl.pallas_call(
        _post_body,
        grid=(grid,),
        in_specs=[
            pl.BlockSpec((blk, H), lambda i: (i, 0)),
            pl.BlockSpec((blk, H), lambda i: (i, 0)),
            pl.BlockSpec((blk, H), lambda i: (i, 0)),
            pl.BlockSpec((blk, 1), lambda i: (i, 0)),
            pl.BlockSpec((blk, H), lambda i: (i, 0)),
            pl.BlockSpec((blk, F), lambda i: (i, 0)),
            pl.BlockSpec((1, H), lambda i: (0, 0)),
            pl.BlockSpec((H, F), lambda i: (0, 0)),
        ],
        out_specs=[
            pl.BlockSpec((blk, F), lambda i: (i, 0)),
            pl.BlockSpec((blk, H), lambda i: (i, 0)),
        ],
        out_shape=[
            jax.ShapeDtypeStruct((N, F), jnp.float32),
            jax.ShapeDtypeStruct((N, H), jnp.float32),
        ],
    )(a0, a1, xws, dinv, bm, cm, gb, v)


def kernel(x, hidden_state, edge_index, gcn_W, gcn_b, b_matrix, v_matrix,
           c_matrix):
    src = edge_index[0]
    dst = edge_index[1]
    pad = E_PAD - E
    srcm = jnp.concatenate([src, jnp.zeros((pad,), jnp.int32)]).reshape(
        NCHUNK, CHUNK)
    dstm = jnp.concatenate([dst, jnp.full((pad,), N, jnp.int32)]).reshape(
        NCHUNK, CHUNK)

    ones16 = jnp.ones((CHUNK, 16), jnp.float32)
    zeros16 = jnp.zeros((AGG_ROWS, 16), jnp.float32)
    zer = jnp.zeros((AGG_ROWS, H), jnp.float32)

    deg = _deg_kernel(dstm, ones16, zeros16)
    d0 = deg[0, :N, 0:1]
    d1 = deg[1, :N, 0:1]

    w1 = gcn_W[:F]
    w2 = gcn_W[F:]
    xws, dinv = _pre_call(x, hidden_state, w1, w2, d0, d1)

    agg = _agg_kernel(xws, srcm, dstm, zer)

    o, nh = _post_call(agg[0, :N], agg[1, :N], xws, dinv, b_matrix, c_matrix,
                       gcn_b.reshape(1, H), v_matrix)
    return (o, nh)


# speedup vs baseline: 11.4620x; 11.4620x over previous
import functools

import jax
import jax.numpy as jnp
from jax import lax
from jax.experimental import pallas as pl
from jax.experimental.pallas import tpu as pltpu
from jax.experimental.pallas import tpu_sc as plsc

N = 10000
F = 128
H = 128
E = 320000

NC = 2
NS = 16
NW = NC * NS
CHUNK = 128
E_PAD = 327680
NCHUNK = E_PAD // CHUNK
CPW = NCHUNK // NW
AGG_ROWS = 10112
RPT = AGG_ROWS // NS

_mesh = plsc.VectorSubcoreMesh(
    core_axis_name="c", subcore_axis_name="s", num_cores=NC, num_subcores=NS)


@functools.partial(
    pl.kernel,
    out_type=jax.ShapeDtypeStruct((NC, AGG_ROWS, 16), jnp.float32),
    mesh=_mesh,
    scratch_types=[
        pltpu.VMEM_SHARED((AGG_ROWS, 16), jnp.float32),
        pltpu.VMEM((CPW, 2, CHUNK), jnp.int32),
        pltpu.VMEM((CHUNK, 16), jnp.float32),
        pltpu.SemaphoreType.DMA,
    ],
    compiler_params=pltpu.CompilerParams(use_tc_tiling_on_sc=False),
)
def _deg_kernel(edg, ones16, zeros16, out, degm, idx_all, ones_v, sem):
    c = lax.axis_index("c")
    s = lax.axis_index("s")
    wid = c * NS + s
    base = wid * CPW
    pltpu.sync_copy(zeros16.at[pl.ds(s * RPT, RPT)], degm.at[pl.ds(s * RPT, RPT)])
    pltpu.sync_copy(ones16, ones_v)
    pltpu.sync_copy(edg.at[pl.ds(base, CPW)], idx_all)
    plsc.subcore_barrier()

    for g in range(0, CPW, 16):
        for b in range(16):
            pltpu.async_copy(
                ones_v, degm.at[idx_all.at[g + b, 1]], sem, add=True)
        for b in range(16):
            pltpu.make_async_copy(ones_v, degm.at[idx_all.at[0, 1]], sem).wait()

    plsc.subcore_barrier()
    pltpu.sync_copy(degm.at[pl.ds(s * RPT, RPT)], out.at[c, pl.ds(s * RPT, RPT)])


@functools.partial(
    pl.kernel,
    out_type=jax.ShapeDtypeStruct((NC, AGG_ROWS, H), jnp.float32),
    mesh=_mesh,
    scratch_types=[
        pltpu.VMEM_SHARED((AGG_ROWS, H), jnp.float32),
        pltpu.VMEM((4, 2, CHUNK), jnp.int32),
        pltpu.VMEM((2, CHUNK, H), jnp.float32),
        [pltpu.SemaphoreType.DMA] * 4,
        [pltpu.SemaphoreType.DMA] * 2,
    ],
)
def _agg_kernel(xws, edg, zer, out, agg, eb, rows, isems, gsems):
    c = lax.axis_index("c")
    s = lax.axis_index("s")
    wid = c * NS + s
    base = wid * CPW
    pltpu.sync_copy(zer.at[pl.ds(s * RPT, RPT)], agg.at[pl.ds(s * RPT, RPT)])
    plsc.subcore_barrier()

    def fetch_idx(j, ib):
        pltpu.async_copy(edg.at[base + j], eb.at[ib], isems[ib])

    def wait_idx(ib):
        pltpu.make_async_copy(edg.at[base], eb.at[ib], isems[ib]).wait()

    def start_gather(j_ib, gb):
        pltpu.async_copy(xws.at[eb.at[j_ib, 0]], rows.at[gb], gsems[gb])

    def wait_gather(gb):
        pltpu.make_async_copy(
            xws.at[eb.at[0, 0]], rows.at[gb], gsems[gb]).wait()

    for j0 in range(4):
        fetch_idx(j0, j0)
    for j0 in range(2):
        wait_idx(j0)
        start_gather(j0, j0)

    @pl.loop(0, CPW, step=4)
    def _step(j):
        for b in range(4):
            jj = j + b
            gb = b % 2
            wait_gather(gb)
            pltpu.sync_copy(rows.at[gb], agg.at[eb.at[b, 1]], add=True)

            @pl.when(jj + 4 < CPW)
            def _fetch():
                fetch_idx(jj + 4, b)

            @pl.when(jj + 2 < CPW)
            def _next_gather():
                wait_idx((b + 2) % 4)
                start_gather((b + 2) % 4, gb)

    plsc.subcore_barrier()
    pltpu.sync_copy(agg.at[pl.ds(s * RPT, RPT)], out.at[c, pl.ds(s * RPT, RPT)])


def _pre_body(x_ref, h_ref, w1_ref, w2_ref, d0_ref, d1_ref, xws_ref, dinv_ref):
    xw = (jnp.dot(x_ref[...], w1_ref[...], preferred_element_type=jnp.float32)
          + jnp.dot(h_ref[...], w2_ref[...], preferred_element_type=jnp.float32))
    dinv = lax.rsqrt(d0_ref[...] + d1_ref[...] + 1.0)
    dinv_ref[...] = dinv
    xws_ref[...] = xw * dinv


def _pre_call(x, h, w1, w2, d0, d1):
    blk = 1000
    grid = N // blk
    return pl.pallas_call(
        _pre_body,
        grid=(grid,),
        in_specs=[
            pl.BlockSpec((blk, F), lambda i: (i, 0)),
            pl.BlockSpec((blk, H), lambda i: (i, 0)),
            pl.BlockSpec((F, H), lambda i: (0, 0)),
            pl.BlockSpec((H, H), lambda i: (0, 0)),
            pl.BlockSpec((blk, 1), lambda i: (i, 0)),
            pl.BlockSpec((blk, 1), lambda i: (i, 0)),
        ],
        out_specs=[
            pl.BlockSpec((blk, H), lambda i: (i, 0)),
            pl.BlockSpec((blk, 1), lambda i: (i, 0)),
        ],
        out_shape=[
            jax.ShapeDtypeStruct((N, H), jnp.float32),
            jax.ShapeDtypeStruct((N, 1), jnp.float32),
        ],
    )(x, h, w1, w2, d0, d1)


def _post_body(a0_ref, a1_ref, xws_ref, dinv_ref, bm_ref, cm_ref, gb_ref,
               v_ref, o_ref, nh_ref):
    conv = (a0_ref[...] + a1_ref[...] + xws_ref[...]) * dinv_ref[...] + gb_ref[...]
    nh = jax.nn.sigmoid(bm_ref[...] + conv)
    nh_ref[...] = nh
    o_ref[...] = cm_ref[...] + jnp.dot(nh, v_ref[...],
                                       preferred_element_type=jnp.float32)


def _post_call(a0, a1, xws, dinv, bm, cm, gb, v):
    blk = 1000
    grid = N // blk
    return pl.pallas_call(
        _post_body,
        grid=(grid,),
        in_specs=[
            pl.BlockSpec((blk, H), lambda i: (i, 0)),
            pl.BlockSpec((blk, H), lambda i: (i, 0)),
            pl.BlockSpec((blk, H), lambda i: (i, 0)),
            pl.BlockSpec((blk, 1), lambda i: (i, 0)),
            pl.BlockSpec((blk, H), lambda i: (i, 0)),
            pl.BlockSpec((blk, F), lambda i: (i, 0)),
            pl.BlockSpec((1, H), lambda i: (0, 0)),
            pl.BlockSpec((H, F), lambda i: (0, 0)),
        ],
        out_specs=[
            pl.BlockSpec((blk, F), lambda i: (i, 0)),
            pl.BlockSpec((blk, H), lambda i: (i, 0)),
        ],
        out_shape=[
            jax.ShapeDtypeStruct((N, F), jnp.float32),
            jax.ShapeDtypeStruct((N, H), jnp.float32),
        ],
    )(a0, a1, xws, dinv, bm, cm, gb, v)


def kernel(x, hidden_state, edge_index, gcn_W, gcn_b, b_matrix, v_matrix,
           c_matrix):
    src = edge_index[0]
    dst = edge_index[1]
    pad = E_PAD - E
    srcm = jnp.concatenate([src, jnp.zeros((pad,), jnp.int32)]).reshape(
        NCHUNK, CHUNK)
    dstm = jnp.concatenate([dst, jnp.full((pad,), N, jnp.int32)]).reshape(
        NCHUNK, CHUNK)
    edg = jnp.stack([srcm, dstm], axis=1)

    ones16 = jnp.ones((CHUNK, 16), jnp.float32)
    zeros16 = jnp.zeros((AGG_ROWS, 16), jnp.float32)
    zer = jnp.zeros((AGG_ROWS, H), jnp.float32)

    deg = _deg_kernel(edg, ones16, zeros16)
    d0 = deg[0, :N, 0:1]
    d1 = deg[1, :N, 0:1]

    w1 = gcn_W[:F]
    w2 = gcn_W[F:]
    xws, dinv = _pre_call(x, hidden_state, w1, w2, d0, d1)

    agg = _agg_kernel(xws, edg, zer)

    o, nh = _post_call(agg[0, :N], agg[1, :N], xws, dinv, b_matrix, c_matrix,
                       gcn_b.reshape(1, H), v_matrix)
    return (o, nh)
